# baseline (speedup 1.0000x reference)
import functools

import jax
import jax.numpy as jnp
from jax import lax
from jax.experimental import pallas as pl
from jax.experimental.pallas import tpu as pltpu
from jax.experimental.pallas import tpu_sc as plsc

_BINS = 64
_EPS = 1e-08
_NC, _NS, _L = 2, 16, 16
_NW = _NC * _NS
_W = 512
_R = 64
_PIECES = 512 // _R
_U = 16
_ACC = 4 * _BINS * _L


def _sc_hist_body(gen_hbm, tgt_hbm, out_hbm, buf0, buf1, acc, acc2, accr,
                  sem0, sem1):
    w = lax.axis_index("s") * _NC + lax.axis_index("c")

    zeros = jnp.zeros((_L,), jnp.float32)

    def zero_body(i, _):
        acc[pl.ds(i * _L, _L)] = zeros
        acc2[pl.ds(i * _L, _L)] = zeros
        return 0

    lax.fori_loop(0, _ACC // _L, zero_body, 0)

    lane = lax.iota(jnp.int32, _L)
    ones = jnp.ones((_L,), jnp.float32)

    def compute_piece(buf, off_vec):
        @plsc.parallel_loop(0, _R * _W // _L // 2, unroll=_U // 2)
        def _(vi):
            for u, a in ((0, acc), (1, acc2)):
                v2 = vi * 2 + u
                r = v2 >> 5
                c = (v2 & 31) * _L
                v = buf[r, pl.ds(c, _L)]
                y = v * 0.5 + 1.5
                y = jnp.minimum(jnp.maximum(y, 1.0), 1.9990234375)
                bits = plsc.bitcast(y, jnp.int32)
                b16 = jnp.bitwise_and(
                    lax.shift_right_logical(bits, 13), (_BINS - 1) * _L)
                plsc.addupdate_scatter(a, [b16 + off_vec], ones)

    npc = 2 * _PIECES

    def gen_piece(p):
        return gen_hbm.at[w, 1 + (p >> 3), pl.ds((p & 7) * _R, _R), :]

    def tgt_piece(p):
        return tgt_hbm.at[w, 1 + (p >> 3), pl.ds((p & 7) * _R, _R), :]

    def off_for(p, hbase):
        return lane + (hbase + (p >> 3)) * _BINS * _L

    pltpu.async_copy(gen_piece(0), buf0, sem0)

    def pair_body(j, _):
        pltpu.async_copy(tgt_piece(j), buf1, sem1)
        pltpu.make_async_copy(gen_piece(0), buf0, sem0).wait()
        compute_piece(buf0, off_for(j, 0))

        @pl.when(j < npc - 1)
        def _():
            pltpu.async_copy(gen_piece(j + 1), buf0, sem0)

        pltpu.make_async_copy(tgt_piece(0), buf1, sem1).wait()
        compute_piece(buf1, off_for(j, 2))
        return 0

    lax.fori_loop(0, npc, pair_body, 0)

    iota16 = lane * _L

    def red_body(g, _):
        base = g * (_L * _L)
        s = jnp.zeros((_L,), jnp.float32)
        for l in range(_L):
            s = s + plsc.load_gather(acc, [base + iota16 + l])
            s = s + plsc.load_gather(acc2, [base + iota16 + l])
        accr[pl.ds(g * _L, _L)] = s
        return 0

    lax.fori_loop(0, 4 * _BINS // _L, red_body, 0)

    pltpu.sync_copy(accr, out_hbm.at[w])


@functools.lru_cache(maxsize=None)
def _build_sc_hist():
    mesh = plsc.VectorSubcoreMesh(
        core_axis_name="c", subcore_axis_name="s",
        num_cores=_NC, num_subcores=_NS)
    return pl.kernel(
        _sc_hist_body,
        out_type=jax.ShapeDtypeStruct((_NW, 4 * _BINS), jnp.float32),
        mesh=mesh,
        scratch_types=[
            pltpu.VMEM((_R, _W), jnp.float32),
            pltpu.VMEM((_R, _W), jnp.float32),
            pltpu.VMEM((_ACC,), jnp.float32),
            pltpu.VMEM((_ACC,), jnp.float32),
            pltpu.VMEM((4 * _BINS,), jnp.float32),
            pltpu.SemaphoreType.DMA,
            pltpu.SemaphoreType.DMA,
        ],
        compiler_params=pltpu.CompilerParams(needs_layout_passes=False),
    )


def _finalize_body(x_ref, o_ref):
    x = x_ref[...]
    s = jnp.sum(x, axis=0, keepdims=True)

    def _norm(hh):
        hh = hh + _EPS
        return hh / jnp.sum(hh)

    g1 = _norm(s[:, 0:_BINS])
    g2 = _norm(s[:, _BINS:2 * _BINS])
    t1 = _norm(s[:, 2 * _BINS:3 * _BINS])
    t2 = _norm(s[:, 3 * _BINS:4 * _BINS])
    kl = (jnp.sum(t1 * (jnp.log(t1) - jnp.log(g1 + _EPS)))
          + jnp.sum(t2 * (jnp.log(t2) - jnp.log(g2 + _EPS))))
    o_ref[0, 0] = kl / (2 * _BINS)


def kernel(generated, target):
    parts = _build_sc_hist()(generated, target)
    out = pl.pallas_call(
        _finalize_body,
        out_shape=jax.ShapeDtypeStruct((1, 1), jnp.float32),
        out_specs=pl.BlockSpec(memory_space=pltpu.SMEM),
    )(parts)
    return out.reshape(())


# speedup vs baseline: 1.0390x; 1.0390x over previous
import functools

import jax
import jax.numpy as jnp
from jax import lax
from jax.experimental import pallas as pl
from jax.experimental.pallas import tpu as pltpu
from jax.experimental.pallas import tpu_sc as plsc

_BINS = 64
_EPS = 1e-08
_NC, _NS, _L = 2, 16, 16
_NW = _NC * _NS
_W = 512
_R = 64
_PIECES = 512 // _R
_U = 16
_ACC = 4 * _BINS * _L


def _sc_hist_body(gen_hbm, tgt_hbm, out_hbm, buf0, buf1, acc, accr, sem0, sem1):
    w = lax.axis_index("s") * _NC + lax.axis_index("c")

    zeros = jnp.zeros((_L,), jnp.float32)

    def zero_body(i, _):
        acc[pl.ds(i * _L, _L)] = zeros
        return 0

    lax.fori_loop(0, _ACC // _L, zero_body, 0)

    lane = lax.iota(jnp.int32, _L)
    ones = jnp.ones((_L,), jnp.float32)

    def compute_piece(buf, off_vec):
        @plsc.parallel_loop(0, _R * _W // _L, unroll=_U)
        def _(vi):
            r = vi >> 5
            c = (vi & 31) * _L
            v = buf[r, pl.ds(c, _L)]
            y = v * 0.5 + 1.5
            y = jnp.minimum(jnp.maximum(y, 1.0), 1.9990234375)
            bits = plsc.bitcast(y, jnp.int32)
            b16 = jnp.bitwise_and(
                lax.shift_right_logical(bits, 13), (_BINS - 1) * _L)
            plsc.addupdate_scatter(acc, [b16 + off_vec], ones)

    npc = 2 * _PIECES

    def gen_piece(p):
        return gen_hbm.at[w, 1 + (p >> 3), pl.ds((p & 7) * _R, _R), :]

    def tgt_piece(p):
        return tgt_hbm.at[w, 1 + (p >> 3), pl.ds((p & 7) * _R, _R), :]

    def off_for(p, hbase):
        return lane + (hbase + (p >> 3)) * _BINS * _L

    pltpu.async_copy(gen_piece(0), buf0, sem0)

    def pair_body(j, _):
        pltpu.async_copy(tgt_piece(j), buf1, sem1)
        pltpu.make_async_copy(gen_piece(0), buf0, sem0).wait()
        compute_piece(buf0, off_for(j, 0))

        @pl.when(j < npc - 1)
        def _():
            pltpu.async_copy(gen_piece(j + 1), buf0, sem0)

        pltpu.make_async_copy(tgt_piece(0), buf1, sem1).wait()
        compute_piece(buf1, off_for(j, 2))
        return 0

    lax.fori_loop(0, npc, pair_body, 0)

    iota16 = lane * _L

    def red_body(g, _):
        base = g * (_L * _L)
        s = jnp.zeros((_L,), jnp.float32)
        for l in range(_L):
            s = s + plsc.load_gather(acc, [base + iota16 + l])
        accr[pl.ds(g * _L, _L)] = s
        return 0

    lax.fori_loop(0, 4 * _BINS // _L, red_body, 0)

    pltpu.sync_copy(accr, out_hbm.at[w])


@functools.lru_cache(maxsize=None)
def _build_sc_hist():
    mesh = plsc.VectorSubcoreMesh(
        core_axis_name="c", subcore_axis_name="s",
        num_cores=_NC, num_subcores=_NS)
    return pl.kernel(
        _sc_hist_body,
        out_type=jax.ShapeDtypeStruct((_NW, 4 * _BINS), jnp.float32),
        mesh=mesh,
        scratch_types=[
            pltpu.VMEM((_R, _W), jnp.float32),
            pltpu.VMEM((_R, _W), jnp.float32),
            pltpu.VMEM((_ACC,), jnp.float32),
            pltpu.VMEM((4 * _BINS,), jnp.float32),
            pltpu.SemaphoreType.DMA,
            pltpu.SemaphoreType.DMA,
        ],
        compiler_params=pltpu.CompilerParams(needs_layout_passes=False),
    )


def _finalize_body(x_ref, o_ref):
    x = x_ref[...]
    s = jnp.sum(x, axis=0, keepdims=True)

    def _norm(hh):
        hh = hh + _EPS
        return hh / jnp.sum(hh)

    g1 = _norm(s[:, 0:_BINS])
    g2 = _norm(s[:, _BINS:2 * _BINS])
    t1 = _norm(s[:, 2 * _BINS:3 * _BINS])
    t2 = _norm(s[:, 3 * _BINS:4 * _BINS])
    kl = (jnp.sum(t1 * (jnp.log(t1) - jnp.log(g1 + _EPS)))
          + jnp.sum(t2 * (jnp.log(t2) - jnp.log(g2 + _EPS))))
    o_ref[0, 0] = kl / (2 * _BINS)


def kernel(generated, target):
    parts = _build_sc_hist()(generated, target)
    out = pl.pallas_call(
        _finalize_body,
        out_shape=jax.ShapeDtypeStruct((1, 1), jnp.float32),
        out_specs=pl.BlockSpec(memory_space=pltpu.SMEM),
    )(parts)
    return out.reshape(())
